# Initial kernel scaffold; baseline (speedup 1.0000x reference)
import functools

import jax
import jax.numpy as jnp
from jax import lax
from jax.experimental import pallas as pl
from jax.experimental.pallas import tpu as pltpu
from jax.experimental.pallas import tpu_sc as plsc

_N = 10000
_E = 320000
_D_IN = 128
_H1 = 256
_H2 = 32
_C = 16

_NC = 2
_NS = 16
_NW = _NC * _NS
_CHUNK = 128
_CPW = -(-_E // (_CHUNK * _NW))
_EPAD = _CPW * _CHUNK * _NW
_RPS = _N // _NS

_BLK = 1000


@functools.cache
def _make_agg(d):
    mesh = plsc.VectorSubcoreMesh(core_axis_name="c", subcore_axis_name="s")
    out = jax.ShapeDtypeStruct((_N, d), jnp.float32)

    @functools.partial(
        pl.kernel,
        out_type=(out, out),
        mesh=mesh,
        scratch_types=[
            pltpu.VMEM((_CPW, _CHUNK), jnp.int32),
            pltpu.VMEM((_CPW, _CHUNK), jnp.int32),
            pltpu.VMEM((_CHUNK, d), jnp.float32),
            pltpu.VMEM_SHARED((_N + 8, d), jnp.float32),
            pltpu.SemaphoreType.DMA,
        ],
    )
    def agg(vals, srci, dsti, zeros, out0, out1, sidx, didx, rows, acc, sem):
        c = lax.axis_index("c")
        s = lax.axis_index("s")
        w = c * _NS + s
        sl = pl.ds(s * _RPS, _RPS)
        pltpu.sync_copy(zeros.at[sl], acc.at[sl])
        pltpu.sync_copy(srci.at[w], sidx)
        pltpu.sync_copy(dsti.at[w], didx)
        plsc.subcore_barrier()

        @pl.loop(0, _CPW)
        def _(t):
            pltpu.async_copy(vals.at[sidx.at[t]], rows, sem).wait()
            pltpu.sync_copy(rows, acc.at[didx.at[t]], add=True)

        plsc.subcore_barrier()

        @pl.when(c == 0)
        def _():
            pltpu.sync_copy(acc.at[sl], out0.at[sl])

        @pl.when(c == 1)
        def _():
            pltpu.sync_copy(acc.at[sl], out1.at[sl])

    return agg


def _dinv(da, db):
    return lax.rsqrt(da[:, :1] + db[:, :1] + 1.0)


def _row_spec(d):
    return pl.BlockSpec((_BLK, d), lambda i: (i, 0))


def _full_spec(r, c):
    return pl.BlockSpec((r, c), lambda i: (0, 0))


def _k_scale(x, da, db):
    def body(x_ref, da_ref, db_ref, o_ref):
        o_ref[...] = x_ref[...] * _dinv(da_ref[...], db_ref[...])

    return pl.pallas_call(
        body,
        grid=(_N // _BLK,),
        in_specs=[_row_spec(_D_IN), _row_spec(16), _row_spec(16)],
        out_specs=_row_spec(_D_IN),
        out_shape=jax.ShapeDtypeStruct((_N, _D_IN), jnp.float32),
    )(x, da, db)


def _k_layer12(a0, a1, xs, da, db, W1, b1, W2):
    def body(a0_ref, a1_ref, xs_ref, da_ref, db_ref, w1_ref, b1_ref, w2_ref, o_ref):
        di = _dinv(da_ref[...], db_ref[...])
        agg = (a0_ref[...] + a1_ref[...] + xs_ref[...]) * di
        h = jnp.dot(agg, w1_ref[...], preferred_element_type=jnp.float32)
        h = jnp.maximum(h + b1_ref[...], 0.0)
        o_ref[...] = jnp.dot(h, w2_ref[...], preferred_element_type=jnp.float32) * di

    return pl.pallas_call(
        body,
        grid=(_N // _BLK,),
        in_specs=[_row_spec(_D_IN), _row_spec(_D_IN), _row_spec(_D_IN),
                  _row_spec(16), _row_spec(16),
                  _full_spec(_D_IN, _H1), _full_spec(1, _H1), _full_spec(_H1, _H2)],
        out_specs=_row_spec(_H2),
        out_shape=jax.ShapeDtypeStruct((_N, _H2), jnp.float32),
    )(a0, a1, xs, da, db, W1, b1.reshape(1, _H1), W2)


def _k_layer23(a0, a1, t2s, da, db, b2, W3):
    def body(a0_ref, a1_ref, t_ref, da_ref, db_ref, b2_ref, w3_ref, o_ref):
        di = _dinv(da_ref[...], db_ref[...])
        h = (a0_ref[...] + a1_ref[...] + t_ref[...]) * di + b2_ref[...]
        h = jnp.maximum(h, 0.0)
        o_ref[...] = jnp.dot(h, w3_ref[...], preferred_element_type=jnp.float32) * di

    return pl.pallas_call(
        body,
        grid=(_N // _BLK,),
        in_specs=[_row_spec(_H2), _row_spec(_H2), _row_spec(_H2),
                  _row_spec(16), _row_spec(16),
                  _full_spec(1, _H2), _full_spec(_H2, _C)],
        out_specs=_row_spec(_C),
        out_shape=jax.ShapeDtypeStruct((_N, _C), jnp.float32),
    )(a0, a1, t2s, da, db, b2.reshape(1, _H2), W3)


def _k_out(a0, a1, t3s, da, db, b3):
    def body(a0_ref, a1_ref, t_ref, da_ref, db_ref, b3_ref, o_ref):
        di = _dinv(da_ref[...], db_ref[...])
        o_ref[...] = (a0_ref[...] + a1_ref[...] + t_ref[...]) * di + b3_ref[...]

    return pl.pallas_call(
        body,
        grid=(_N // _BLK,),
        in_specs=[_row_spec(_C), _row_spec(_C), _row_spec(_C),
                  _row_spec(16), _row_spec(16), _full_spec(1, _C)],
        out_specs=_row_spec(_C),
        out_shape=jax.ShapeDtypeStruct((_N, _C), jnp.float32),
    )(a0, a1, t3s, da, db, b3.reshape(1, _C))


def kernel(x, edge_index, W1, b1, W2, b2, W3, b3):
    ei = edge_index.astype(jnp.int32)
    pad = _EPAD - _E
    srcp = jnp.concatenate([ei[0], jnp.zeros((pad,), jnp.int32)])
    dstp = jnp.concatenate([ei[1], jnp.full((pad,), _N, jnp.int32)])
    srcp = srcp.reshape(_NW, _CPW, _CHUNK)
    dstp = dstp.reshape(_NW, _CPW, _CHUNK)

    ones16 = jnp.ones((_N, 16), jnp.float32)
    z16 = jnp.zeros((_N, 16), jnp.float32)
    z32 = jnp.zeros((_N, _H2), jnp.float32)
    z128 = jnp.zeros((_N, _D_IN), jnp.float32)

    agg16 = _make_agg(16)
    agg32 = _make_agg(_H2)
    agg128 = _make_agg(_D_IN)

    da, db = agg16(ones16, srcp, dstp, z16)
    xs = _k_scale(x, da, db)
    a0, a1 = agg128(xs, srcp, dstp, z128)
    t2s = _k_layer12(a0, a1, xs, da, db, W1, b1, W2)
    c0, c1 = agg32(t2s, srcp, dstp, z32)
    t3s = _k_layer23(c0, c1, t2s, da, db, b2, W3)
    d0, d1 = agg16(t3s, srcp, dstp, z16)
    return _k_out(d0, d1, t3s, da, db, b3)


# speedup vs baseline: 17.1246x; 17.1246x over previous
import functools

import jax
import jax.numpy as jnp
from jax import lax
from jax.experimental import pallas as pl
from jax.experimental.pallas import tpu as pltpu
from jax.experimental.pallas import tpu_sc as plsc

_N = 10000
_E = 320000
_D_IN = 128
_DH = _D_IN // 2
_H1 = 256
_H2 = 32
_C = 16

_NC = 2
_NS = 16
_NW = _NC * _NS
_CHUNK = 128
_CPW = 80
_CPS = 160
_EPAD = _CPW * _CHUNK * _NW
_RPS = 624
_TAIL = _N - _NS * _RPS

_BLK = 1000

_SC_PARAMS = pltpu.CompilerParams(use_tc_tiling_on_sc=False)


def _mesh():
    return plsc.VectorSubcoreMesh(core_axis_name="c", subcore_axis_name="s")


def _init_acc(zeros, acc, s, sl, tl):
    pltpu.sync_copy(zeros.at[sl], acc.at[sl])

    @pl.when(s == _NS - 1)
    def _():
        pltpu.sync_copy(zeros.at[tl], acc.at[tl])


def _pipelined_edge_loop(vals, sidx, didx, rows0, rows1, acc,
                         semg0, semg1, sems0, sems1, cpw):
    pltpu.async_copy(vals.at[sidx.at[0]], rows0, semg0)
    pltpu.async_copy(vals.at[sidx.at[1]], rows1, semg1)
    plsc.subcore_barrier()

    @pl.loop(0, cpw // 2)
    def _(u):
        t0 = 2 * u
        t1 = t0 + 1
        pltpu.make_async_copy(vals.at[sidx.at[t0]], rows0, semg0).wait()
        pltpu.async_copy(rows0, acc.at[didx.at[t0]], sems0, add=True)
        pltpu.make_async_copy(vals.at[sidx.at[t1]], rows1, semg1).wait()
        pltpu.async_copy(rows1, acc.at[didx.at[t1]], sems1, add=True)

        @pl.when(t0 + 2 < cpw)
        def _():
            pltpu.make_async_copy(rows0, acc.at[didx.at[t0]], sems0).wait()
            pltpu.async_copy(vals.at[sidx.at[t0 + 2]], rows0, semg0)
            pltpu.make_async_copy(rows1, acc.at[didx.at[t1]], sems1).wait()
            pltpu.async_copy(vals.at[sidx.at[t1 + 2]], rows1, semg1)

    pltpu.make_async_copy(rows0, acc.at[didx.at[cpw - 2]], sems0).wait()
    pltpu.make_async_copy(rows1, acc.at[didx.at[cpw - 1]], sems1).wait()
    plsc.subcore_barrier()


@functools.cache
def _make_agg(d):
    out = jax.ShapeDtypeStruct((_N, d), jnp.float32)

    @functools.partial(
        pl.kernel,
        out_type=(out, out),
        mesh=_mesh(),
        compiler_params=_SC_PARAMS,
        scratch_types=[
            pltpu.VMEM((_CPW, _CHUNK), jnp.int32),
            pltpu.VMEM((_CPW, _CHUNK), jnp.int32),
            pltpu.VMEM((_CHUNK, d), jnp.float32),
            pltpu.VMEM((_CHUNK, d), jnp.float32),
            pltpu.VMEM_SHARED((_N + 8, d), jnp.float32),
            pltpu.SemaphoreType.DMA,
            pltpu.SemaphoreType.DMA,
            pltpu.SemaphoreType.DMA,
            pltpu.SemaphoreType.DMA,
        ],
    )
    def agg(vals, srci, dsti, zeros, out0, out1,
            sidx, didx, rows0, rows1, acc, semg0, semg1, sems0, sems1):
        c = lax.axis_index("c")
        s = lax.axis_index("s")
        w = c * _NS + s
        sl = pl.ds(s * _RPS, _RPS)
        tl = pl.ds(_NS * _RPS, _TAIL)
        _init_acc(zeros, acc, s, sl, tl)
        pltpu.sync_copy(srci.at[w], sidx)
        pltpu.sync_copy(dsti.at[w], didx)
        _pipelined_edge_loop(vals, sidx, didx, rows0, rows1, acc,
                             semg0, semg1, sems0, sems1, _CPW)

        @pl.when(c == 0)
        def _():
            pltpu.sync_copy(acc.at[sl], out0.at[sl])

            @pl.when(s == _NS - 1)
            def _():
                pltpu.sync_copy(acc.at[tl], out0.at[tl])

        @pl.when(c == 1)
        def _():
            pltpu.sync_copy(acc.at[sl], out1.at[sl])

            @pl.when(s == _NS - 1)
            def _():
                pltpu.sync_copy(acc.at[tl], out1.at[tl])

    return agg


def _make_agg_cols():
    out = jax.ShapeDtypeStruct((_NC, _N, _DH), jnp.float32)

    @functools.partial(
        pl.kernel,
        out_type=out,
        mesh=_mesh(),
        compiler_params=_SC_PARAMS,
        scratch_types=[
            pltpu.VMEM((_CPS, _CHUNK), jnp.int32),
            pltpu.VMEM((_CPS, _CHUNK), jnp.int32),
            pltpu.VMEM((_CHUNK, _DH), jnp.float32),
            pltpu.VMEM((_CHUNK, _DH), jnp.float32),
            pltpu.VMEM_SHARED((_N + 8, _DH), jnp.float32),
            pltpu.SemaphoreType.DMA,
            pltpu.SemaphoreType.DMA,
            pltpu.SemaphoreType.DMA,
            pltpu.SemaphoreType.DMA,
        ],
    )
    def agg(vals, srci, dsti, zeros, outx,
            sidx, didx, rows0, rows1, acc, semg0, semg1, sems0, sems1):
        c = lax.axis_index("c")
        s = lax.axis_index("s")
        sl = pl.ds(s * _RPS, _RPS)
        tl = pl.ds(_NS * _RPS, _TAIL)
        _init_acc(zeros, acc, s, sl, tl)
        pltpu.sync_copy(srci.at[s], sidx)
        pltpu.sync_copy(dsti.at[s], didx)
        myvals = vals.at[c]
        myout = outx.at[c]
        _pipelined_edge_loop(myvals, sidx, didx, rows0, rows1, acc,
                             semg0, semg1, sems0, sems1, _CPS)
        pltpu.sync_copy(acc.at[sl], myout.at[sl])

        @pl.when(s == _NS - 1)
        def _():
            pltpu.sync_copy(acc.at[tl], myout.at[tl])

    return agg


def _dinv(da, db):
    return lax.rsqrt(da[:, :1] + db[:, :1] + 1.0)


def _row_spec(d):
    return pl.BlockSpec((_BLK, d), lambda i: (i, 0))


def _half_spec():
    return pl.BlockSpec((_NC, _BLK, _DH), lambda i: (0, i, 0))


def _full_spec(r, c):
    return pl.BlockSpec((r, c), lambda i: (0, 0))


def _k_scale(x, da, db):
    def body(x_ref, da_ref, db_ref, o_ref):
        di = _dinv(da_ref[...], db_ref[...])
        xs = x_ref[...] * di
        o_ref[0] = xs[:, :_DH]
        o_ref[1] = xs[:, _DH:]

    return pl.pallas_call(
        body,
        grid=(_N // _BLK,),
        in_specs=[_row_spec(_D_IN), _row_spec(16), _row_spec(16)],
        out_specs=_half_spec(),
        out_shape=jax.ShapeDtypeStruct((_NC, _N, _DH), jnp.float32),
    )(x, da, db)


def _k_layer12(ax, xs, da, db, W1, b1, W2):
    def body(ax_ref, xs_ref, da_ref, db_ref, w1a_ref, w1b_ref, b1_ref,
             w2_ref, o_ref):
        di = _dinv(da_ref[...], db_ref[...])
        agg_lo = (ax_ref[0] + xs_ref[0]) * di
        agg_hi = (ax_ref[1] + xs_ref[1]) * di
        h = (jnp.dot(agg_lo, w1a_ref[...], preferred_element_type=jnp.float32)
             + jnp.dot(agg_hi, w1b_ref[...], preferred_element_type=jnp.float32))
        h = jnp.maximum(h + b1_ref[...], 0.0)
        o_ref[...] = jnp.dot(h, w2_ref[...], preferred_element_type=jnp.float32) * di

    return pl.pallas_call(
        body,
        grid=(_N // _BLK,),
        in_specs=[_half_spec(), _half_spec(),
                  _row_spec(16), _row_spec(16),
                  _full_spec(_DH, _H1), _full_spec(_DH, _H1),
                  _full_spec(1, _H1), _full_spec(_H1, _H2)],
        out_specs=_row_spec(_H2),
        out_shape=jax.ShapeDtypeStruct((_N, _H2), jnp.float32),
    )(ax, xs, da, db, W1[:_DH], W1[_DH:], b1.reshape(1, _H1), W2)


def _k_layer23(a0, a1, t2s, da, db, b2, W3):
    def body(a0_ref, a1_ref, t_ref, da_ref, db_ref, b2_ref, w3_ref, o_ref):
        di = _dinv(da_ref[...], db_ref[...])
        h = (a0_ref[...] + a1_ref[...] + t_ref[...]) * di + b2_ref[...]
        h = jnp.maximum(h, 0.0)
        o_ref[...] = jnp.dot(h, w3_ref[...], preferred_element_type=jnp.float32) * di

    return pl.pallas_call(
        body,
        grid=(_N // _BLK,),
        in_specs=[_row_spec(_H2), _row_spec(_H2), _row_spec(_H2),
                  _row_spec(16), _row_spec(16),
                  _full_spec(1, _H2), _full_spec(_H2, _C)],
        out_specs=_row_spec(_C),
        out_shape=jax.ShapeDtypeStruct((_N, _C), jnp.float32),
    )(a0, a1, t2s, da, db, b2.reshape(1, _H2), W3)


def _k_out(a0, a1, t3s, da, db, b3):
    def body(a0_ref, a1_ref, t_ref, da_ref, db_ref, b3_ref, o_ref):
        di = _dinv(da_ref[...], db_ref[...])
        o_ref[...] = (a0_ref[...] + a1_ref[...] + t_ref[...]) * di + b3_ref[...]

    return pl.pallas_call(
        body,
        grid=(_N // _BLK,),
        in_specs=[_row_spec(_C), _row_spec(_C), _row_spec(_C),
                  _row_spec(16), _row_spec(16), _full_spec(1, _C)],
        out_specs=_row_spec(_C),
        out_shape=jax.ShapeDtypeStruct((_N, _C), jnp.float32),
    )(a0, a1, t3s, da, db, b3.reshape(1, _C))


def kernel(x, edge_index, W1, b1, W2, b2, W3, b3):
    ei = edge_index.astype(jnp.int32)
    pad = _EPAD - _E
    srcp = jnp.concatenate([ei[0], jnp.zeros((pad,), jnp.int32)])
    dstp = jnp.concatenate([ei[1], jnp.full((pad,), _N, jnp.int32)])
    srcw = srcp.reshape(_NW, _CPW, _CHUNK)
    dstw = dstp.reshape(_NW, _CPW, _CHUNK)
    srcs = srcp.reshape(_NS, _CPS, _CHUNK)
    dsts = dstp.reshape(_NS, _CPS, _CHUNK)

    ones16 = jnp.ones((_N, 16), jnp.float32)
    z16 = jnp.zeros((_N, 16), jnp.float32)
    z32 = jnp.zeros((_N, _H2), jnp.float32)
    z64 = jnp.zeros((_N, _DH), jnp.float32)

    agg16 = _make_agg(16)
    agg32 = _make_agg(_H2)
    agg64 = _make_agg_cols()

    da, db = agg16(ones16, srcw, dstw, z16)
    xs = _k_scale(x, da, db)
    ax = agg64(xs, srcs, dsts, z64)
    t2s = _k_layer12(ax, xs, da, db, W1, b1, W2)
    c0, c1 = agg32(t2s, srcw, dstw, z32)
    t3s = _k_layer23(c0, c1, t2s, da, db, b2, W3)
    d0, d1 = agg16(t3s, srcw, dstw, z16)
    return _k_out(d0, d1, t3s, da, db, b3)


# speedup vs baseline: 19.4448x; 1.1355x over previous
import functools

import jax
import jax.numpy as jnp
from jax import lax
from jax.experimental import pallas as pl
from jax.experimental.pallas import tpu as pltpu
from jax.experimental.pallas import tpu_sc as plsc

_N = 10000
_E = 320000
_D_IN = 128
_DH = _D_IN // 2
_H1 = 256
_H2 = 32
_C = 16

_NC = 2
_NS = 16
_NW = _NC * _NS
_CHUNK = 128
_CPW = 80
_CPS = 160
_EPAD = _CPW * _CHUNK * _NW
_RPS = 624
_TAIL = _N - _NS * _RPS

_BLK = 1000

_SC_PARAMS = pltpu.CompilerParams(use_tc_tiling_on_sc=False)


def _mesh():
    return plsc.VectorSubcoreMesh(core_axis_name="c", subcore_axis_name="s")


def _init_acc(zeros, acc, s, sl, tl):
    pltpu.sync_copy(zeros.at[sl], acc.at[sl])

    @pl.when(s == _NS - 1)
    def _():
        pltpu.sync_copy(zeros.at[tl], acc.at[tl])


def _pipelined_edge_loop(vals, sidx, didx, rows0, rows1, acc,
                         semg0, semg1, sems0, sems1, cpw):
    pltpu.async_copy(vals.at[sidx.at[0]], rows0, semg0)
    pltpu.async_copy(vals.at[sidx.at[1]], rows1, semg1)
    plsc.subcore_barrier()

    @pl.loop(0, cpw // 2)
    def _(u):
        t0 = 2 * u
        t1 = t0 + 1
        pltpu.make_async_copy(vals.at[sidx.at[t0]], rows0, semg0).wait()
        pltpu.async_copy(rows0, acc.at[didx.at[t0]], sems0, add=True)
        pltpu.make_async_copy(vals.at[sidx.at[t1]], rows1, semg1).wait()
        pltpu.async_copy(rows1, acc.at[didx.at[t1]], sems1, add=True)

        @pl.when(t0 + 2 < cpw)
        def _():
            pltpu.make_async_copy(rows0, acc.at[didx.at[t0]], sems0).wait()
            pltpu.async_copy(vals.at[sidx.at[t0 + 2]], rows0, semg0)
            pltpu.make_async_copy(rows1, acc.at[didx.at[t1]], sems1).wait()
            pltpu.async_copy(vals.at[sidx.at[t1 + 2]], rows1, semg1)

    pltpu.make_async_copy(rows0, acc.at[didx.at[cpw - 2]], sems0).wait()
    pltpu.make_async_copy(rows1, acc.at[didx.at[cpw - 1]], sems1).wait()
    plsc.subcore_barrier()


@functools.cache
def _make_agg(d):
    out = jax.ShapeDtypeStruct((_N, d), jnp.float32)

    @functools.partial(
        pl.kernel,
        out_type=(out, out),
        mesh=_mesh(),
        compiler_params=_SC_PARAMS,
        scratch_types=[
            pltpu.VMEM((_CPW, _CHUNK), jnp.int32),
            pltpu.VMEM((_CPW, _CHUNK), jnp.int32),
            pltpu.VMEM((_CHUNK, d), jnp.float32),
            pltpu.VMEM((_CHUNK, d), jnp.float32),
            pltpu.VMEM_SHARED((_N + 8, d), jnp.float32),
            pltpu.SemaphoreType.DMA,
            pltpu.SemaphoreType.DMA,
            pltpu.SemaphoreType.DMA,
            pltpu.SemaphoreType.DMA,
        ],
    )
    def agg(vals, srci, dsti, zeros, out0, out1,
            sidx, didx, rows0, rows1, acc, semg0, semg1, sems0, sems1):
        c = lax.axis_index("c")
        s = lax.axis_index("s")
        w = c * _NS + s
        sl = pl.ds(s * _RPS, _RPS)
        tl = pl.ds(_NS * _RPS, _TAIL)
        _init_acc(zeros, acc, s, sl, tl)
        pltpu.sync_copy(srci.at[w], sidx)
        pltpu.sync_copy(dsti.at[w], didx)
        _pipelined_edge_loop(vals, sidx, didx, rows0, rows1, acc,
                             semg0, semg1, sems0, sems1, _CPW)

        @pl.when(c == 0)
        def _():
            pltpu.sync_copy(acc.at[sl], out0.at[sl])

            @pl.when(s == _NS - 1)
            def _():
                pltpu.sync_copy(acc.at[tl], out0.at[tl])

        @pl.when(c == 1)
        def _():
            pltpu.sync_copy(acc.at[sl], out1.at[sl])

            @pl.when(s == _NS - 1)
            def _():
                pltpu.sync_copy(acc.at[tl], out1.at[tl])

    return agg


def _make_deg():
    out = jax.ShapeDtypeStruct((_N, 16), jnp.float32)

    @functools.partial(
        pl.kernel,
        out_type=(out, out),
        mesh=_mesh(),
        compiler_params=_SC_PARAMS,
        scratch_types=[
            pltpu.VMEM((_CPW, _CHUNK), jnp.int32),
            pltpu.VMEM((_CHUNK, 16), jnp.float32),
            pltpu.VMEM_SHARED((_N + 8, 16), jnp.float32),
            pltpu.SemaphoreType.DMA,
        ],
    )
    def deg(ones_hbm, dsti, zeros, out0, out1, didx, ones, acc, sem):
        c = lax.axis_index("c")
        s = lax.axis_index("s")
        w = c * _NS + s
        sl = pl.ds(s * _RPS, _RPS)
        tl = pl.ds(_NS * _RPS, _TAIL)
        _init_acc(zeros, acc, s, sl, tl)
        pltpu.sync_copy(ones_hbm, ones)
        pltpu.sync_copy(dsti.at[w], didx)
        plsc.subcore_barrier()

        @pl.loop(0, _CPW // 8)
        def _(g):
            base = g * 8
            for j in range(8):
                pltpu.async_copy(ones, acc.at[didx.at[base + j]], sem, add=True)
            for j in range(8):
                pltpu.make_async_copy(ones, acc.at[didx.at[base + j]], sem).wait()

        plsc.subcore_barrier()

        @pl.when(c == 0)
        def _():
            pltpu.sync_copy(acc.at[sl], out0.at[sl])

            @pl.when(s == _NS - 1)
            def _():
                pltpu.sync_copy(acc.at[tl], out0.at[tl])

        @pl.when(c == 1)
        def _():
            pltpu.sync_copy(acc.at[sl], out1.at[sl])

            @pl.when(s == _NS - 1)
            def _():
                pltpu.sync_copy(acc.at[tl], out1.at[tl])

    return deg


def _make_agg_cols():
    out = jax.ShapeDtypeStruct((_NC, _N, _DH), jnp.float32)

    @functools.partial(
        pl.kernel,
        out_type=out,
        mesh=_mesh(),
        compiler_params=_SC_PARAMS,
        scratch_types=[
            pltpu.VMEM((_CPS, _CHUNK), jnp.int32),
            pltpu.VMEM((_CPS, _CHUNK), jnp.int32),
            pltpu.VMEM((_CHUNK, _DH), jnp.float32),
            pltpu.VMEM((_CHUNK, _DH), jnp.float32),
            pltpu.VMEM_SHARED((_N + 8, _DH), jnp.float32),
            pltpu.SemaphoreType.DMA,
            pltpu.SemaphoreType.DMA,
            pltpu.SemaphoreType.DMA,
            pltpu.SemaphoreType.DMA,
        ],
    )
    def agg(vals, srci, dsti, zeros, outx,
            sidx, didx, rows0, rows1, acc, semg0, semg1, sems0, sems1):
        c = lax.axis_index("c")
        s = lax.axis_index("s")
        sl = pl.ds(s * _RPS, _RPS)
        tl = pl.ds(_NS * _RPS, _TAIL)
        _init_acc(zeros, acc, s, sl, tl)
        pltpu.sync_copy(srci.at[s], sidx)
        pltpu.sync_copy(dsti.at[s], didx)
        myvals = vals.at[c]
        myout = outx.at[c]
        _pipelined_edge_loop(myvals, sidx, didx, rows0, rows1, acc,
                             semg0, semg1, sems0, sems1, _CPS)
        pltpu.sync_copy(acc.at[sl], myout.at[sl])

        @pl.when(s == _NS - 1)
        def _():
            pltpu.sync_copy(acc.at[tl], myout.at[tl])

    return agg


def _dinv(da, db):
    return lax.rsqrt(da[:, :1] + db[:, :1] + 1.0)


def _row_spec(d):
    return pl.BlockSpec((_BLK, d), lambda i: (i, 0))


def _half_spec():
    return pl.BlockSpec((_NC, _BLK, _DH), lambda i: (0, i, 0))


def _full_spec(r, c):
    return pl.BlockSpec((r, c), lambda i: (0, 0))


def _k_scale(x, da, db):
    def body(x_ref, da_ref, db_ref, o_ref):
        di = _dinv(da_ref[...], db_ref[...])
        xs = x_ref[...] * di
        o_ref[0] = xs[:, :_DH]
        o_ref[1] = xs[:, _DH:]

    return pl.pallas_call(
        body,
        grid=(_N // _BLK,),
        in_specs=[_row_spec(_D_IN), _row_spec(16), _row_spec(16)],
        out_specs=_half_spec(),
        out_shape=jax.ShapeDtypeStruct((_NC, _N, _DH), jnp.float32),
    )(x, da, db)


def _k_layer12(ax, xs, da, db, W1, b1, W2):
    def body(ax_ref, xs_ref, da_ref, db_ref, w1a_ref, w1b_ref, b1_ref,
             w2_ref, o_ref):
        di = _dinv(da_ref[...], db_ref[...])
        agg_lo = (ax_ref[0] + xs_ref[0]) * di
        agg_hi = (ax_ref[1] + xs_ref[1]) * di
        h = (jnp.dot(agg_lo, w1a_ref[...], preferred_element_type=jnp.float32)
             + jnp.dot(agg_hi, w1b_ref[...], preferred_element_type=jnp.float32))
        h = jnp.maximum(h + b1_ref[...], 0.0)
        o_ref[...] = jnp.dot(h, w2_ref[...], preferred_element_type=jnp.float32) * di

    return pl.pallas_call(
        body,
        grid=(_N // _BLK,),
        in_specs=[_half_spec(), _half_spec(),
                  _row_spec(16), _row_spec(16),
                  _full_spec(_DH, _H1), _full_spec(_DH, _H1),
                  _full_spec(1, _H1), _full_spec(_H1, _H2)],
        out_specs=_row_spec(_H2),
        out_shape=jax.ShapeDtypeStruct((_N, _H2), jnp.float32),
    )(ax, xs, da, db, W1[:_DH], W1[_DH:], b1.reshape(1, _H1), W2)


def _k_layer23(a0, a1, t2s, da, db, b2, W3):
    def body(a0_ref, a1_ref, t_ref, da_ref, db_ref, b2_ref, w3_ref, o_ref):
        di = _dinv(da_ref[...], db_ref[...])
        h = (a0_ref[...] + a1_ref[...] + t_ref[...]) * di + b2_ref[...]
        h = jnp.maximum(h, 0.0)
        o_ref[...] = jnp.dot(h, w3_ref[...], preferred_element_type=jnp.float32) * di

    return pl.pallas_call(
        body,
        grid=(_N // _BLK,),
        in_specs=[_row_spec(_H2), _row_spec(_H2), _row_spec(_H2),
                  _row_spec(16), _row_spec(16),
                  _full_spec(1, _H2), _full_spec(_H2, _C)],
        out_specs=_row_spec(_C),
        out_shape=jax.ShapeDtypeStruct((_N, _C), jnp.float32),
    )(a0, a1, t2s, da, db, b2.reshape(1, _H2), W3)


def _k_out(a0, a1, t3s, da, db, b3):
    def body(a0_ref, a1_ref, t_ref, da_ref, db_ref, b3_ref, o_ref):
        di = _dinv(da_ref[...], db_ref[...])
        o_ref[...] = (a0_ref[...] + a1_ref[...] + t_ref[...]) * di + b3_ref[...]

    return pl.pallas_call(
        body,
        grid=(_N // _BLK,),
        in_specs=[_row_spec(_C), _row_spec(_C), _row_spec(_C),
                  _row_spec(16), _row_spec(16), _full_spec(1, _C)],
        out_specs=_row_spec(_C),
        out_shape=jax.ShapeDtypeStruct((_N, _C), jnp.float32),
    )(a0, a1, t3s, da, db, b3.reshape(1, _C))


def kernel(x, edge_index, W1, b1, W2, b2, W3, b3):
    ei = edge_index.astype(jnp.int32)
    pad = _EPAD - _E
    srcp = jnp.concatenate([ei[0], jnp.zeros((pad,), jnp.int32)])
    dstp = jnp.concatenate(
        [ei[1], _N + (jnp.arange(pad, dtype=jnp.int32) % 8)])
    srcw = srcp.reshape(_CPW, _NW, _CHUNK).transpose(1, 0, 2)
    dstw = dstp.reshape(_CPW, _NW, _CHUNK).transpose(1, 0, 2)
    srcs = srcp.reshape(_CPS, _NS, _CHUNK).transpose(1, 0, 2)
    dsts = dstp.reshape(_CPS, _NS, _CHUNK).transpose(1, 0, 2)

    ones_blk = jnp.ones((_CHUNK, 16), jnp.float32)
    z16 = jnp.zeros((_N, 16), jnp.float32)
    z32 = jnp.zeros((_N, _H2), jnp.float32)
    z64 = jnp.zeros((_N, _DH), jnp.float32)

    degk = _make_deg()
    agg16 = _make_agg(16)
    agg32 = _make_agg(_H2)
    agg64 = _make_agg_cols()

    da, db = degk(ones_blk, dstw, z16)
    xs = _k_scale(x, da, db)
    ax = agg64(xs, srcs, dsts, z64)
    t2s = _k_layer12(ax, xs, da, db, W1, b1, W2)
    c0, c1 = agg32(t2s, srcw, dstw, z32)
    t3s = _k_layer23(c0, c1, t2s, da, db, b2, W3)
    d0, d1 = agg16(t3s, srcw, dstw, z16)
    return _k_out(d0, d1, t3s, da, db, b3)


# speedup vs baseline: 24.0930x; 1.2390x over previous
import functools

import jax
import jax.numpy as jnp
from jax import lax
from jax.experimental import pallas as pl
from jax.experimental.pallas import tpu as pltpu
from jax.experimental.pallas import tpu_sc as plsc

_N = 10000
_E = 320000
_D_IN = 128
_DH = _D_IN // 2
_H1 = 256
_H2 = 32
_C = 16

_NC = 2
_NS = 16
_NW = _NC * _NS
_CHUNK = 128
_CPW = 80
_CPS = 160
_EPAD = _CPW * _CHUNK * _NW
_RPS = 624
_TAIL = _N - _NS * _RPS

_BLK = 1000

_SC_PARAMS = pltpu.CompilerParams(use_tc_tiling_on_sc=False)


def _mesh():
    return plsc.VectorSubcoreMesh(core_axis_name="c", subcore_axis_name="s")


def _init_acc(zeros, acc, s, sl, tl):
    pltpu.sync_copy(zeros.at[sl], acc.at[sl])

    @pl.when(s == _NS - 1)
    def _():
        pltpu.sync_copy(zeros.at[tl], acc.at[tl])


def _pipelined_edge_loop(vals, sidx, didx, rows0, rows1, acc,
                         semg0, semg1, sems0, sems1, cpw):
    plsc.subcore_barrier()
    pltpu.async_copy(vals.at[sidx.at[0]], rows0, semg0)
    pltpu.async_copy(vals.at[sidx.at[1]], rows1, semg1)

    @pl.loop(0, cpw // 2)
    def _(u):
        t0 = 2 * u
        t1 = t0 + 1
        pltpu.make_async_copy(vals.at[sidx.at[t0]], rows0, semg0).wait()
        pltpu.async_copy(rows0, acc.at[didx.at[t0]], sems0, add=True)
        pltpu.make_async_copy(vals.at[sidx.at[t1]], rows1, semg1).wait()
        pltpu.async_copy(rows1, acc.at[didx.at[t1]], sems1, add=True)

        @pl.when(t0 + 2 < cpw)
        def _():
            pltpu.make_async_copy(rows0, acc.at[didx.at[t0]], sems0).wait()
            pltpu.async_copy(vals.at[sidx.at[t0 + 2]], rows0, semg0)
            pltpu.make_async_copy(rows1, acc.at[didx.at[t1]], sems1).wait()
            pltpu.async_copy(vals.at[sidx.at[t1 + 2]], rows1, semg1)

    pltpu.make_async_copy(rows0, acc.at[didx.at[cpw - 2]], sems0).wait()
    pltpu.make_async_copy(rows1, acc.at[didx.at[cpw - 1]], sems1).wait()
    plsc.subcore_barrier()


@functools.cache
def _make_agg(d):
    out = jax.ShapeDtypeStruct((_N, d), jnp.float32)

    @functools.partial(
        pl.kernel,
        out_type=(out, out),
        mesh=_mesh(),
        compiler_params=_SC_PARAMS,
        scratch_types=[
            pltpu.VMEM((_CPW, _CHUNK), jnp.int32),
            pltpu.VMEM((_CPW, _CHUNK), jnp.int32),
            pltpu.VMEM((_CHUNK, d), jnp.float32),
            pltpu.VMEM((_CHUNK, d), jnp.float32),
            pltpu.VMEM_SHARED((_N + 8, d), jnp.float32),
            pltpu.VMEM_SHARED((_N + 8, d), jnp.float32),
            pltpu.SemaphoreType.DMA,
            pltpu.SemaphoreType.DMA,
            pltpu.SemaphoreType.DMA,
            pltpu.SemaphoreType.DMA,
        ],
    )
    def agg(vals, srci, dsti, zeros, out0, out1,
            sidx, didx, rows0, rows1, acc, srcbuf, semg0, semg1, sems0, sems1):
        c = lax.axis_index("c")
        s = lax.axis_index("s")
        w = c * _NS + s
        sl = pl.ds(s * _RPS, _RPS)
        tl = pl.ds(_NS * _RPS, _TAIL)
        _init_acc(zeros, acc, s, sl, tl)
        pltpu.sync_copy(vals.at[sl], srcbuf.at[sl])

        @pl.when(s == _NS - 1)
        def _():
            pltpu.sync_copy(vals.at[tl], srcbuf.at[tl])

        pltpu.sync_copy(srci.at[w], sidx)
        pltpu.sync_copy(dsti.at[w], didx)
        _pipelined_edge_loop(srcbuf, sidx, didx, rows0, rows1, acc,
                             semg0, semg1, sems0, sems1, _CPW)

        @pl.when(c == 0)
        def _():
            pltpu.sync_copy(acc.at[sl], out0.at[sl])

            @pl.when(s == _NS - 1)
            def _():
                pltpu.sync_copy(acc.at[tl], out0.at[tl])

        @pl.when(c == 1)
        def _():
            pltpu.sync_copy(acc.at[sl], out1.at[sl])

            @pl.when(s == _NS - 1)
            def _():
                pltpu.sync_copy(acc.at[tl], out1.at[tl])

    return agg


def _make_deg():
    out = jax.ShapeDtypeStruct((_N, 16), jnp.float32)

    @functools.partial(
        pl.kernel,
        out_type=(out, out),
        mesh=_mesh(),
        compiler_params=_SC_PARAMS,
        scratch_types=[
            pltpu.VMEM((_CPW, _CHUNK), jnp.int32),
            pltpu.VMEM((_CHUNK, 16), jnp.float32),
            pltpu.VMEM_SHARED((_N + 8, 16), jnp.float32),
            pltpu.SemaphoreType.DMA,
        ],
    )
    def deg(ones_hbm, dsti, zeros, out0, out1, didx, ones, acc, sem):
        c = lax.axis_index("c")
        s = lax.axis_index("s")
        w = c * _NS + s
        sl = pl.ds(s * _RPS, _RPS)
        tl = pl.ds(_NS * _RPS, _TAIL)
        _init_acc(zeros, acc, s, sl, tl)
        pltpu.sync_copy(ones_hbm, ones)
        pltpu.sync_copy(dsti.at[w], didx)
        plsc.subcore_barrier()

        @pl.loop(0, _CPW // 8)
        def _(g):
            base = g * 8
            for j in range(8):
                pltpu.async_copy(ones, acc.at[didx.at[base + j]], sem, add=True)
            for j in range(8):
                pltpu.make_async_copy(ones, acc.at[didx.at[base + j]], sem).wait()

        plsc.subcore_barrier()

        @pl.when(c == 0)
        def _():
            pltpu.sync_copy(acc.at[sl], out0.at[sl])

            @pl.when(s == _NS - 1)
            def _():
                pltpu.sync_copy(acc.at[tl], out0.at[tl])

        @pl.when(c == 1)
        def _():
            pltpu.sync_copy(acc.at[sl], out1.at[sl])

            @pl.when(s == _NS - 1)
            def _():
                pltpu.sync_copy(acc.at[tl], out1.at[tl])

    return deg


def _make_agg_cols():
    out = jax.ShapeDtypeStruct((_NC, _N, _DH), jnp.float32)

    @functools.partial(
        pl.kernel,
        out_type=out,
        mesh=_mesh(),
        compiler_params=_SC_PARAMS,
        scratch_types=[
            pltpu.VMEM((_CPS, _CHUNK), jnp.int32),
            pltpu.VMEM((_CPS, _CHUNK), jnp.int32),
            pltpu.VMEM((_CHUNK, _DH), jnp.float32),
            pltpu.VMEM((_CHUNK, _DH), jnp.float32),
            pltpu.VMEM_SHARED((_N + 8, _DH), jnp.float32),
            pltpu.SemaphoreType.DMA,
            pltpu.SemaphoreType.DMA,
            pltpu.SemaphoreType.DMA,
            pltpu.SemaphoreType.DMA,
        ],
    )
    def agg(vals, srci, dsti, zeros, outx,
            sidx, didx, rows0, rows1, acc, semg0, semg1, sems0, sems1):
        c = lax.axis_index("c")
        s = lax.axis_index("s")
        sl = pl.ds(s * _RPS, _RPS)
        tl = pl.ds(_NS * _RPS, _TAIL)
        _init_acc(zeros, acc, s, sl, tl)
        pltpu.sync_copy(srci.at[s], sidx)
        pltpu.sync_copy(dsti.at[s], didx)
        myvals = vals.at[c]
        myout = outx.at[c]
        _pipelined_edge_loop(myvals, sidx, didx, rows0, rows1, acc,
                             semg0, semg1, sems0, sems1, _CPS)
        pltpu.sync_copy(acc.at[sl], myout.at[sl])

        @pl.when(s == _NS - 1)
        def _():
            pltpu.sync_copy(acc.at[tl], myout.at[tl])

    return agg


def _dinv(da, db):
    return lax.rsqrt(da[:, :1] + db[:, :1] + 1.0)


def _row_spec(d):
    return pl.BlockSpec((_BLK, d), lambda i: (i, 0))


def _half_spec():
    return pl.BlockSpec((_NC, _BLK, _DH), lambda i: (0, i, 0))


def _full_spec(r, c):
    return pl.BlockSpec((r, c), lambda i: (0, 0))


def _k_scale(x, da, db):
    def body(x_ref, da_ref, db_ref, o_ref):
        di = _dinv(da_ref[...], db_ref[...])
        xs = x_ref[...] * di
        o_ref[0] = xs[:, :_DH]
        o_ref[1] = xs[:, _DH:]

    return pl.pallas_call(
        body,
        grid=(_N // _BLK,),
        in_specs=[_row_spec(_D_IN), _row_spec(16), _row_spec(16)],
        out_specs=_half_spec(),
        out_shape=jax.ShapeDtypeStruct((_NC, _N, _DH), jnp.float32),
    )(x, da, db)


def _k_layer12(ax, xs, da, db, W1, b1, W2):
    def body(ax_ref, xs_ref, da_ref, db_ref, w1a_ref, w1b_ref, b1_ref,
             w2_ref, o_ref):
        di = _dinv(da_ref[...], db_ref[...])
        agg_lo = (ax_ref[0] + xs_ref[0]) * di
        agg_hi = (ax_ref[1] + xs_ref[1]) * di
        h = (jnp.dot(agg_lo, w1a_ref[...], preferred_element_type=jnp.float32)
             + jnp.dot(agg_hi, w1b_ref[...], preferred_element_type=jnp.float32))
        h = jnp.maximum(h + b1_ref[...], 0.0)
        o_ref[...] = jnp.dot(h, w2_ref[...], preferred_element_type=jnp.float32) * di

    return pl.pallas_call(
        body,
        grid=(_N // _BLK,),
        in_specs=[_half_spec(), _half_spec(),
                  _row_spec(16), _row_spec(16),
                  _full_spec(_DH, _H1), _full_spec(_DH, _H1),
                  _full_spec(1, _H1), _full_spec(_H1, _H2)],
        out_specs=_row_spec(_H2),
        out_shape=jax.ShapeDtypeStruct((_N, _H2), jnp.float32),
    )(ax, xs, da, db, W1[:_DH], W1[_DH:], b1.reshape(1, _H1), W2)


def _k_layer23(a0, a1, t2s, da, db, b2, W3):
    def body(a0_ref, a1_ref, t_ref, da_ref, db_ref, b2_ref, w3_ref, o_ref):
        di = _dinv(da_ref[...], db_ref[...])
        h = (a0_ref[...] + a1_ref[...] + t_ref[...]) * di + b2_ref[...]
        h = jnp.maximum(h, 0.0)
        o_ref[...] = jnp.dot(h, w3_ref[...], preferred_element_type=jnp.float32) * di

    return pl.pallas_call(
        body,
        grid=(_N // _BLK,),
        in_specs=[_row_spec(_H2), _row_spec(_H2), _row_spec(_H2),
                  _row_spec(16), _row_spec(16),
                  _full_spec(1, _H2), _full_spec(_H2, _C)],
        out_specs=_row_spec(_C),
        out_shape=jax.ShapeDtypeStruct((_N, _C), jnp.float32),
    )(a0, a1, t2s, da, db, b2.reshape(1, _H2), W3)


def _k_out(a0, a1, t3s, da, db, b3):
    def body(a0_ref, a1_ref, t_ref, da_ref, db_ref, b3_ref, o_ref):
        di = _dinv(da_ref[...], db_ref[...])
        o_ref[...] = (a0_ref[...] + a1_ref[...] + t_ref[...]) * di + b3_ref[...]

    return pl.pallas_call(
        body,
        grid=(_N // _BLK,),
        in_specs=[_row_spec(_C), _row_spec(_C), _row_spec(_C),
                  _row_spec(16), _row_spec(16), _full_spec(1, _C)],
        out_specs=_row_spec(_C),
        out_shape=jax.ShapeDtypeStruct((_N, _C), jnp.float32),
    )(a0, a1, t3s, da, db, b3.reshape(1, _C))


def kernel(x, edge_index, W1, b1, W2, b2, W3, b3):
    ei = edge_index.astype(jnp.int32)
    pad = _EPAD - _E
    srcp = jnp.concatenate([ei[0], jnp.zeros((pad,), jnp.int32)])
    dstp = jnp.concatenate(
        [ei[1], _N + (jnp.arange(pad, dtype=jnp.int32) % 8)])
    srcw = srcp.reshape(_CPW, _NW, _CHUNK).transpose(1, 0, 2)
    dstw = dstp.reshape(_CPW, _NW, _CHUNK).transpose(1, 0, 2)
    srcs = srcp.reshape(_CPS, _NS, _CHUNK).transpose(1, 0, 2)
    dsts = dstp.reshape(_CPS, _NS, _CHUNK).transpose(1, 0, 2)

    ones_blk = jnp.ones((_CHUNK, 16), jnp.float32)
    z16 = jnp.zeros((_N, 16), jnp.float32)
    z32 = jnp.zeros((_N, _H2), jnp.float32)
    z64 = jnp.zeros((_N, _DH), jnp.float32)

    degk = _make_deg()
    agg16 = _make_agg(16)
    agg32 = _make_agg(_H2)
    agg64 = _make_agg_cols()

    da, db = degk(ones_blk, dstw, z16)
    xs = _k_scale(x, da, db)
    ax = agg64(xs, srcs, dsts, z64)
    t2s = _k_layer12(ax, xs, da, db, W1, b1, W2)
    c0, c1 = agg32(t2s, srcw, dstw, z32)
    t3s = _k_layer23(c0, c1, t2s, da, db, b2, W3)
    d0, d1 = agg16(t3s, srcw, dstw, z16)
    return _k_out(d0, d1, t3s, da, db, b3)


# speedup vs baseline: 31.4436x; 1.3051x over previous
import functools

import jax
import jax.numpy as jnp
from jax import lax
from jax.experimental import pallas as pl
from jax.experimental.pallas import tpu as pltpu
from jax.experimental.pallas import tpu_sc as plsc

_N = 10000
_E = 320000
_D_IN = 128
_DQ = _D_IN // 4
_H1 = 256
_H2 = 32
_C = 16

_NC = 2
_NS = 16
_NW = _NC * _NS
_CHUNK = 128
_CPW = 80
_CPS = 160
_EPAD = _CPW * _CHUNK * _NW
_RPS = 624
_TAIL = _N - _NS * _RPS

_BLK = 1000

_SC_PARAMS = pltpu.CompilerParams(use_tc_tiling_on_sc=False)


def _mesh():
    return plsc.VectorSubcoreMesh(core_axis_name="c", subcore_axis_name="s")


def _init_acc(zeros, acc, s, sl, tl):
    pltpu.sync_copy(zeros.at[sl], acc.at[sl])

    @pl.when(s == _NS - 1)
    def _():
        pltpu.sync_copy(zeros.at[tl], acc.at[tl])


def _pipelined_edge_loop(vals, sidx, didx, rows0, rows1, acc,
                         semg0, semg1, sems0, sems1, cpw):
    plsc.subcore_barrier()
    pltpu.async_copy(vals.at[sidx.at[0]], rows0, semg0)
    pltpu.async_copy(vals.at[sidx.at[1]], rows1, semg1)

    @pl.loop(0, cpw // 2)
    def _(u):
        t0 = 2 * u
        t1 = t0 + 1
        pltpu.make_async_copy(vals.at[sidx.at[t0]], rows0, semg0).wait()
        pltpu.async_copy(rows0, acc.at[didx.at[t0]], sems0, add=True)
        pltpu.make_async_copy(vals.at[sidx.at[t1]], rows1, semg1).wait()
        pltpu.async_copy(rows1, acc.at[didx.at[t1]], sems1, add=True)

        @pl.when(t0 + 2 < cpw)
        def _():
            pltpu.make_async_copy(rows0, acc.at[didx.at[t0]], sems0).wait()
            pltpu.async_copy(vals.at[sidx.at[t0 + 2]], rows0, semg0)
            pltpu.make_async_copy(rows1, acc.at[didx.at[t1]], sems1).wait()
            pltpu.async_copy(vals.at[sidx.at[t1 + 2]], rows1, semg1)

    pltpu.make_async_copy(rows0, acc.at[didx.at[cpw - 2]], sems0).wait()
    pltpu.make_async_copy(rows1, acc.at[didx.at[cpw - 1]], sems1).wait()
    plsc.subcore_barrier()


@functools.cache
def _make_agg(d):
    out = jax.ShapeDtypeStruct((_N, d), jnp.float32)

    @functools.partial(
        pl.kernel,
        out_type=(out, out),
        mesh=_mesh(),
        compiler_params=_SC_PARAMS,
        scratch_types=[
            pltpu.VMEM((_CPW, _CHUNK), jnp.int32),
            pltpu.VMEM((_CPW, _CHUNK), jnp.int32),
            pltpu.VMEM((_CHUNK, d), jnp.float32),
            pltpu.VMEM((_CHUNK, d), jnp.float32),
            pltpu.VMEM_SHARED((_N + 8, d), jnp.float32),
            pltpu.VMEM_SHARED((_N + 8, d), jnp.float32),
            pltpu.SemaphoreType.DMA,
            pltpu.SemaphoreType.DMA,
            pltpu.SemaphoreType.DMA,
            pltpu.SemaphoreType.DMA,
        ],
    )
    def agg(vals, srci, dsti, zeros, out0, out1,
            sidx, didx, rows0, rows1, acc, srcbuf, semg0, semg1, sems0, sems1):
        c = lax.axis_index("c")
        s = lax.axis_index("s")
        w = c * _NS + s
        sl = pl.ds(s * _RPS, _RPS)
        tl = pl.ds(_NS * _RPS, _TAIL)
        _init_acc(zeros, acc, s, sl, tl)
        pltpu.sync_copy(vals.at[sl], srcbuf.at[sl])

        @pl.when(s == _NS - 1)
        def _():
            pltpu.sync_copy(vals.at[tl], srcbuf.at[tl])

        pltpu.sync_copy(srci.at[w], sidx)
        pltpu.sync_copy(dsti.at[w], didx)
        _pipelined_edge_loop(srcbuf, sidx, didx, rows0, rows1, acc,
                             semg0, semg1, sems0, sems1, _CPW)

        @pl.when(c == 0)
        def _():
            pltpu.sync_copy(acc.at[sl], out0.at[sl])

            @pl.when(s == _NS - 1)
            def _():
                pltpu.sync_copy(acc.at[tl], out0.at[tl])

        @pl.when(c == 1)
        def _():
            pltpu.sync_copy(acc.at[sl], out1.at[sl])

            @pl.when(s == _NS - 1)
            def _():
                pltpu.sync_copy(acc.at[tl], out1.at[tl])

    return agg


def _make_deg():
    out = jax.ShapeDtypeStruct((_N, 16), jnp.float32)

    @functools.partial(
        pl.kernel,
        out_type=(out, out),
        mesh=_mesh(),
        compiler_params=_SC_PARAMS,
        scratch_types=[
            pltpu.VMEM((_CPW, _CHUNK), jnp.int32),
            pltpu.VMEM((_CHUNK, 16), jnp.float32),
            pltpu.VMEM_SHARED((_N + 8, 16), jnp.float32),
            pltpu.SemaphoreType.DMA,
        ],
    )
    def deg(ones_hbm, dsti, zeros, out0, out1, didx, ones, acc, sem):
        c = lax.axis_index("c")
        s = lax.axis_index("s")
        w = c * _NS + s
        sl = pl.ds(s * _RPS, _RPS)
        tl = pl.ds(_NS * _RPS, _TAIL)
        _init_acc(zeros, acc, s, sl, tl)
        pltpu.sync_copy(ones_hbm, ones)
        pltpu.sync_copy(dsti.at[w], didx)
        plsc.subcore_barrier()

        @pl.loop(0, _CPW // 8)
        def _(g):
            base = g * 8
            for j in range(8):
                pltpu.async_copy(ones, acc.at[didx.at[base + j]], sem, add=True)
            for j in range(8):
                pltpu.make_async_copy(ones, acc.at[didx.at[base + j]], sem).wait()

        plsc.subcore_barrier()

        @pl.when(c == 0)
        def _():
            pltpu.sync_copy(acc.at[sl], out0.at[sl])

            @pl.when(s == _NS - 1)
            def _():
                pltpu.sync_copy(acc.at[tl], out0.at[tl])

        @pl.when(c == 1)
        def _():
            pltpu.sync_copy(acc.at[sl], out1.at[sl])

            @pl.when(s == _NS - 1)
            def _():
                pltpu.sync_copy(acc.at[tl], out1.at[tl])

    return deg


def _make_agg_cols():
    out = jax.ShapeDtypeStruct((4, _N, _DQ), jnp.float32)

    @functools.partial(
        pl.kernel,
        out_type=out,
        mesh=_mesh(),
        compiler_params=_SC_PARAMS,
        scratch_types=[
            pltpu.VMEM((_CPS, _CHUNK), jnp.int32),
            pltpu.VMEM((_CPS, _CHUNK), jnp.int32),
            pltpu.VMEM((_CHUNK, _DQ), jnp.float32),
            pltpu.VMEM((_CHUNK, _DQ), jnp.float32),
            pltpu.VMEM_SHARED((_N + 8, _DQ), jnp.float32),
            pltpu.VMEM_SHARED((_N + 8, _DQ), jnp.float32),
            pltpu.SemaphoreType.DMA,
            pltpu.SemaphoreType.DMA,
            pltpu.SemaphoreType.DMA,
            pltpu.SemaphoreType.DMA,
        ],
    )
    def agg(vals, srci, dsti, zeros, outx,
            sidx, didx, rows0, rows1, acc, srcbuf, semg0, semg1, sems0, sems1):
        c = lax.axis_index("c")
        s = lax.axis_index("s")
        sl = pl.ds(s * _RPS, _RPS)
        tl = pl.ds(_NS * _RPS, _TAIL)
        pltpu.sync_copy(srci.at[s], sidx)
        pltpu.sync_copy(dsti.at[s], didx)
        for p in range(2):
            q = c * 2 + p
            myvals = vals.at[q]
            myout = outx.at[q]
            _init_acc(zeros, acc, s, sl, tl)
            pltpu.sync_copy(myvals.at[sl], srcbuf.at[sl])

            @pl.when(s == _NS - 1)
            def _():
                pltpu.sync_copy(myvals.at[tl], srcbuf.at[tl])

            _pipelined_edge_loop(srcbuf, sidx, didx, rows0, rows1, acc,
                                 semg0, semg1, sems0, sems1, _CPS)
            pltpu.sync_copy(acc.at[sl], myout.at[sl])

            @pl.when(s == _NS - 1)
            def _():
                pltpu.sync_copy(acc.at[tl], myout.at[tl])

    return agg


def _dinv(da, db):
    return lax.rsqrt(da[:, :1] + db[:, :1] + 1.0)


def _row_spec(d):
    return pl.BlockSpec((_BLK, d), lambda i: (i, 0))


def _quarter_spec():
    return pl.BlockSpec((4, _BLK, _DQ), lambda i: (0, i, 0))


def _full_spec(r, c):
    return pl.BlockSpec((r, c), lambda i: (0, 0))


def _k_scale(x, da, db):
    def body(x_ref, da_ref, db_ref, o_ref):
        di = _dinv(da_ref[...], db_ref[...])
        xs = x_ref[...] * di
        for q in range(4):
            o_ref[q] = xs[:, q * _DQ:(q + 1) * _DQ]

    return pl.pallas_call(
        body,
        grid=(_N // _BLK,),
        in_specs=[_row_spec(_D_IN), _row_spec(16), _row_spec(16)],
        out_specs=_quarter_spec(),
        out_shape=jax.ShapeDtypeStruct((4, _N, _DQ), jnp.float32),
    )(x, da, db)


def _k_layer12(ax, xs, da, db, W1, b1, W2):
    def body(ax_ref, xs_ref, da_ref, db_ref, w1_ref, b1_ref, w2_ref, o_ref):
        di = _dinv(da_ref[...], db_ref[...])
        h = b1_ref[...]
        for q in range(4):
            agg_q = (ax_ref[q] + xs_ref[q]) * di
            h = h + jnp.dot(agg_q, w1_ref[q],
                            preferred_element_type=jnp.float32)
        h = jnp.maximum(h, 0.0)
        o_ref[...] = jnp.dot(h, w2_ref[...], preferred_element_type=jnp.float32) * di

    return pl.pallas_call(
        body,
        grid=(_N // _BLK,),
        in_specs=[_quarter_spec(), _quarter_spec(),
                  _row_spec(16), _row_spec(16),
                  pl.BlockSpec((4, _DQ, _H1), lambda i: (0, 0, 0)),
                  _full_spec(1, _H1), _full_spec(_H1, _H2)],
        out_specs=_row_spec(_H2),
        out_shape=jax.ShapeDtypeStruct((_N, _H2), jnp.float32),
    )(ax, xs, da, db, W1.reshape(4, _DQ, _H1), b1.reshape(1, _H1), W2)


def _k_layer23(a0, a1, t2s, da, db, b2, W3):
    def body(a0_ref, a1_ref, t_ref, da_ref, db_ref, b2_ref, w3_ref, o_ref):
        di = _dinv(da_ref[...], db_ref[...])
        h = (a0_ref[...] + a1_ref[...] + t_ref[...]) * di + b2_ref[...]
        h = jnp.maximum(h, 0.0)
        o_ref[...] = jnp.dot(h, w3_ref[...], preferred_element_type=jnp.float32) * di

    return pl.pallas_call(
        body,
        grid=(_N // _BLK,),
        in_specs=[_row_spec(_H2), _row_spec(_H2), _row_spec(_H2),
                  _row_spec(16), _row_spec(16),
                  _full_spec(1, _H2), _full_spec(_H2, _C)],
        out_specs=_row_spec(_C),
        out_shape=jax.ShapeDtypeStruct((_N, _C), jnp.float32),
    )(a0, a1, t2s, da, db, b2.reshape(1, _H2), W3)


def _k_out(a0, a1, t3s, da, db, b3):
    def body(a0_ref, a1_ref, t_ref, da_ref, db_ref, b3_ref, o_ref):
        di = _dinv(da_ref[...], db_ref[...])
        o_ref[...] = (a0_ref[...] + a1_ref[...] + t_ref[...]) * di + b3_ref[...]

    return pl.pallas_call(
        body,
        grid=(_N // _BLK,),
        in_specs=[_row_spec(_C), _row_spec(_C), _row_spec(_C),
                  _row_spec(16), _row_spec(16), _full_spec(1, _C)],
        out_specs=_row_spec(_C),
        out_shape=jax.ShapeDtypeStruct((_N, _C), jnp.float32),
    )(a0, a1, t3s, da, db, b3.reshape(1, _C))


def kernel(x, edge_index, W1, b1, W2, b2, W3, b3):
    ei = edge_index.astype(jnp.int32)
    pad = _EPAD - _E
    srcp = jnp.concatenate([ei[0], jnp.zeros((pad,), jnp.int32)])
    dstp = jnp.concatenate(
        [ei[1], _N + (jnp.arange(pad, dtype=jnp.int32) % 8)])
    srcw = srcp.reshape(_CPW, _NW, _CHUNK).transpose(1, 0, 2)
    dstw = dstp.reshape(_CPW, _NW, _CHUNK).transpose(1, 0, 2)
    srcs = srcp.reshape(_CPS, _NS, _CHUNK).transpose(1, 0, 2)
    dsts = dstp.reshape(_CPS, _NS, _CHUNK).transpose(1, 0, 2)

    ones_blk = jnp.ones((_CHUNK, 16), jnp.float32)
    z16 = jnp.zeros((_N, 16), jnp.float32)
    z32 = jnp.zeros((_N, _H2), jnp.float32)

    degk = _make_deg()
    agg16 = _make_agg(16)
    agg32 = _make_agg(_H2)
    agg64 = _make_agg_cols()

    da, db = degk(ones_blk, dstw, z16)
    xs = _k_scale(x, da, db)
    ax = agg64(xs, srcs, dsts, z32)
    t2s = _k_layer12(ax, xs, da, db, W1, b1, W2)
    c0, c1 = agg32(t2s, srcw, dstw, z32)
    t3s = _k_layer23(c0, c1, t2s, da, db, b2, W3)
    d0, d1 = agg16(t3s, srcw, dstw, z16)
    return _k_out(d0, d1, t3s, da, db, b3)


# speedup vs baseline: 32.1978x; 1.0240x over previous
import functools

import jax
import jax.numpy as jnp
from jax import lax
from jax.experimental import pallas as pl
from jax.experimental.pallas import tpu as pltpu
from jax.experimental.pallas import tpu_sc as plsc

_N = 10000
_E = 320000
_D_IN = 128
_DQ = _D_IN // 4
_H1 = 256
_H2 = 32
_C = 16

_NC = 2
_NS = 16
_NW = _NC * _NS
_CHUNK = 128
_CPW = 80
_CPS = 160
_EPAD = _CPW * _CHUNK * _NW
_RPS = 624
_TAIL = _N - _NS * _RPS

_BLK = 1000

_ECH = _E // _CHUNK
_PCH = _EPAD // _CHUNK - _ECH

_SC_PARAMS = pltpu.CompilerParams(use_tc_tiling_on_sc=False)


def _mesh():
    return plsc.VectorSubcoreMesh(core_axis_name="c", subcore_axis_name="s")


def _load_idx(echunks, pchunks, idx_v, base, count):
    nreal = count - _PCH

    @pl.when(base + count <= _ECH)
    def _():
        pltpu.sync_copy(echunks.at[pl.ds(base, count)], idx_v)

    @pl.when(base + count > _ECH)
    def _():
        pltpu.sync_copy(echunks.at[pl.ds(base, nreal)],
                        idx_v.at[pl.ds(0, nreal)])
        pltpu.sync_copy(pchunks, idx_v.at[pl.ds(nreal, _PCH)])


def _init_acc(zeros, acc, s, sl, tl):
    pltpu.sync_copy(zeros.at[sl], acc.at[sl])

    @pl.when(s == _NS - 1)
    def _():
        pltpu.sync_copy(zeros.at[tl], acc.at[tl])


def _pipelined_edge_loop(vals, sidx, didx, rows0, rows1, acc,
                         semg0, semg1, sems0, sems1, cpw):
    plsc.subcore_barrier()
    pltpu.async_copy(vals.at[sidx.at[0]], rows0, semg0)
    pltpu.async_copy(vals.at[sidx.at[1]], rows1, semg1)

    @pl.loop(0, cpw // 2)
    def _(u):
        t0 = 2 * u
        t1 = t0 + 1
        pltpu.make_async_copy(vals.at[sidx.at[t0]], rows0, semg0).wait()
        pltpu.async_copy(rows0, acc.at[didx.at[t0]], sems0, add=True)
        pltpu.make_async_copy(vals.at[sidx.at[t1]], rows1, semg1).wait()
        pltpu.async_copy(rows1, acc.at[didx.at[t1]], sems1, add=True)

        @pl.when(t0 + 2 < cpw)
        def _():
            pltpu.make_async_copy(rows0, acc.at[didx.at[t0]], sems0).wait()
            pltpu.async_copy(vals.at[sidx.at[t0 + 2]], rows0, semg0)
            pltpu.make_async_copy(rows1, acc.at[didx.at[t1]], sems1).wait()
            pltpu.async_copy(vals.at[sidx.at[t1 + 2]], rows1, semg1)

    pltpu.make_async_copy(rows0, acc.at[didx.at[cpw - 2]], sems0).wait()
    pltpu.make_async_copy(rows1, acc.at[didx.at[cpw - 1]], sems1).wait()
    plsc.subcore_barrier()


@functools.cache
def _make_agg(d):
    out = jax.ShapeDtypeStruct((_N, d), jnp.float32)

    @functools.partial(
        pl.kernel,
        out_type=(out, out),
        mesh=_mesh(),
        compiler_params=_SC_PARAMS,
        scratch_types=[
            pltpu.VMEM((_CPW, _CHUNK), jnp.int32),
            pltpu.VMEM((_CPW, _CHUNK), jnp.int32),
            pltpu.VMEM((_CHUNK, d), jnp.float32),
            pltpu.VMEM((_CHUNK, d), jnp.float32),
            pltpu.VMEM_SHARED((_N + 8, d), jnp.float32),
            pltpu.VMEM_SHARED((_N + 8, d), jnp.float32),
            pltpu.SemaphoreType.DMA,
            pltpu.SemaphoreType.DMA,
            pltpu.SemaphoreType.DMA,
            pltpu.SemaphoreType.DMA,
        ],
    )
    def agg(vals, edges, pads, zeros, out0, out1,
            sidx, didx, rows0, rows1, acc, srcbuf, semg0, semg1, sems0, sems1):
        c = lax.axis_index("c")
        s = lax.axis_index("s")
        w = c * _NS + s
        sl = pl.ds(s * _RPS, _RPS)
        tl = pl.ds(_NS * _RPS, _TAIL)
        _init_acc(zeros, acc, s, sl, tl)
        pltpu.sync_copy(vals.at[sl], srcbuf.at[sl])

        @pl.when(s == _NS - 1)
        def _():
            pltpu.sync_copy(vals.at[tl], srcbuf.at[tl])

        _load_idx(edges.at[0], pads.at[0], sidx, w * _CPW, _CPW)
        _load_idx(edges.at[1], pads.at[1], didx, w * _CPW, _CPW)
        _pipelined_edge_loop(srcbuf, sidx, didx, rows0, rows1, acc,
                             semg0, semg1, sems0, sems1, _CPW)

        @pl.when(c == 0)
        def _():
            pltpu.sync_copy(acc.at[sl], out0.at[sl])

            @pl.when(s == _NS - 1)
            def _():
                pltpu.sync_copy(acc.at[tl], out0.at[tl])

        @pl.when(c == 1)
        def _():
            pltpu.sync_copy(acc.at[sl], out1.at[sl])

            @pl.when(s == _NS - 1)
            def _():
                pltpu.sync_copy(acc.at[tl], out1.at[tl])

    return agg


def _make_deg():
    out = jax.ShapeDtypeStruct((_N, 16), jnp.float32)

    @functools.partial(
        pl.kernel,
        out_type=(out, out),
        mesh=_mesh(),
        compiler_params=_SC_PARAMS,
        scratch_types=[
            pltpu.VMEM((_CPW, _CHUNK), jnp.int32),
            pltpu.VMEM((_CHUNK, 16), jnp.float32),
            pltpu.VMEM_SHARED((_N + 8, 16), jnp.float32),
            pltpu.SemaphoreType.DMA,
        ],
    )
    def deg(ones_hbm, edges, pads, zeros, out0, out1, didx, ones, acc, sem):
        c = lax.axis_index("c")
        s = lax.axis_index("s")
        w = c * _NS + s
        sl = pl.ds(s * _RPS, _RPS)
        tl = pl.ds(_NS * _RPS, _TAIL)
        _init_acc(zeros, acc, s, sl, tl)
        pltpu.sync_copy(ones_hbm, ones)
        _load_idx(edges.at[1], pads.at[1], didx, w * _CPW, _CPW)
        plsc.subcore_barrier()

        @pl.loop(0, _CPW // 8)
        def _(g):
            base = g * 8
            for j in range(8):
                pltpu.async_copy(ones, acc.at[didx.at[base + j]], sem, add=True)
            for j in range(8):
                pltpu.make_async_copy(ones, acc.at[didx.at[base + j]], sem).wait()

        plsc.subcore_barrier()

        @pl.when(c == 0)
        def _():
            pltpu.sync_copy(acc.at[sl], out0.at[sl])

            @pl.when(s == _NS - 1)
            def _():
                pltpu.sync_copy(acc.at[tl], out0.at[tl])

        @pl.when(c == 1)
        def _():
            pltpu.sync_copy(acc.at[sl], out1.at[sl])

            @pl.when(s == _NS - 1)
            def _():
                pltpu.sync_copy(acc.at[tl], out1.at[tl])

    return deg


def _make_agg_cols():
    out = jax.ShapeDtypeStruct((4, _N, _DQ), jnp.float32)

    @functools.partial(
        pl.kernel,
        out_type=out,
        mesh=_mesh(),
        compiler_params=_SC_PARAMS,
        scratch_types=[
            pltpu.VMEM((_CPS, _CHUNK), jnp.int32),
            pltpu.VMEM((_CPS, _CHUNK), jnp.int32),
            pltpu.VMEM((_CHUNK, _DQ), jnp.float32),
            pltpu.VMEM((_CHUNK, _DQ), jnp.float32),
            pltpu.VMEM_SHARED((_N + 8, _DQ), jnp.float32),
            pltpu.VMEM_SHARED((_N + 8, _DQ), jnp.float32),
            pltpu.SemaphoreType.DMA,
            pltpu.SemaphoreType.DMA,
            pltpu.SemaphoreType.DMA,
            pltpu.SemaphoreType.DMA,
        ],
    )
    def agg(vals, edges, pads, zeros, outx,
            sidx, didx, rows0, rows1, acc, srcbuf, semg0, semg1, sems0, sems1):
        c = lax.axis_index("c")
        s = lax.axis_index("s")
        sl = pl.ds(s * _RPS, _RPS)
        tl = pl.ds(_NS * _RPS, _TAIL)
        _load_idx(edges.at[0], pads.at[0], sidx, s * _CPS, _CPS)
        _load_idx(edges.at[1], pads.at[1], didx, s * _CPS, _CPS)
        for p in range(2):
            q = c * 2 + p
            myvals = vals.at[q]
            myout = outx.at[q]
            _init_acc(zeros, acc, s, sl, tl)
            pltpu.sync_copy(myvals.at[sl], srcbuf.at[sl])

            @pl.when(s == _NS - 1)
            def _():
                pltpu.sync_copy(myvals.at[tl], srcbuf.at[tl])

            _pipelined_edge_loop(srcbuf, sidx, didx, rows0, rows1, acc,
                                 semg0, semg1, sems0, sems1, _CPS)
            pltpu.sync_copy(acc.at[sl], myout.at[sl])

            @pl.when(s == _NS - 1)
            def _():
                pltpu.sync_copy(acc.at[tl], myout.at[tl])

    return agg


def _dinv(da, db):
    return lax.rsqrt(da[:, :1] + db[:, :1] + 1.0)


def _row_spec(d):
    return pl.BlockSpec((_BLK, d), lambda i: (i, 0))


def _quarter_spec():
    return pl.BlockSpec((4, _BLK, _DQ), lambda i: (0, i, 0))


def _full_spec(r, c):
    return pl.BlockSpec((r, c), lambda i: (0, 0))


def _k_scale(x, da, db):
    def body(x_ref, da_ref, db_ref, o_ref):
        di = _dinv(da_ref[...], db_ref[...])
        xs = x_ref[...] * di
        for q in range(4):
            o_ref[q] = xs[:, q * _DQ:(q + 1) * _DQ]

    return pl.pallas_call(
        body,
        grid=(_N // _BLK,),
        in_specs=[_row_spec(_D_IN), _row_spec(16), _row_spec(16)],
        out_specs=_quarter_spec(),
        out_shape=jax.ShapeDtypeStruct((4, _N, _DQ), jnp.float32),
    )(x, da, db)


def _k_layer12(ax, xs, da, db, W1, b1, W2):
    def body(ax_ref, xs_ref, da_ref, db_ref, w1_ref, b1_ref, w2_ref, o_ref):
        di = _dinv(da_ref[...], db_ref[...])
        h = b1_ref[...]
        for q in range(4):
            agg_q = (ax_ref[q] + xs_ref[q]) * di
            h = h + jnp.dot(agg_q, w1_ref[q],
                            preferred_element_type=jnp.float32)
        h = jnp.maximum(h, 0.0)
        o_ref[...] = jnp.dot(h, w2_ref[...], preferred_element_type=jnp.float32) * di

    return pl.pallas_call(
        body,
        grid=(_N // _BLK,),
        in_specs=[_quarter_spec(), _quarter_spec(),
                  _row_spec(16), _row_spec(16),
                  pl.BlockSpec((4, _DQ, _H1), lambda i: (0, 0, 0)),
                  _full_spec(1, _H1), _full_spec(_H1, _H2)],
        out_specs=_row_spec(_H2),
        out_shape=jax.ShapeDtypeStruct((_N, _H2), jnp.float32),
    )(ax, xs, da, db, W1.reshape(4, _DQ, _H1), b1.reshape(1, _H1), W2)


def _k_layer23(a0, a1, t2s, da, db, b2, W3):
    def body(a0_ref, a1_ref, t_ref, da_ref, db_ref, b2_ref, w3_ref, o_ref):
        di = _dinv(da_ref[...], db_ref[...])
        h = (a0_ref[...] + a1_ref[...] + t_ref[...]) * di + b2_ref[...]
        h = jnp.maximum(h, 0.0)
        o_ref[...] = jnp.dot(h, w3_ref[...], preferred_element_type=jnp.float32) * di

    return pl.pallas_call(
        body,
        grid=(_N // _BLK,),
        in_specs=[_row_spec(_H2), _row_spec(_H2), _row_spec(_H2),
                  _row_spec(16), _row_spec(16),
                  _full_spec(1, _H2), _full_spec(_H2, _C)],
        out_specs=_row_spec(_C),
        out_shape=jax.ShapeDtypeStruct((_N, _C), jnp.float32),
    )(a0, a1, t2s, da, db, b2.reshape(1, _H2), W3)


def _k_out(a0, a1, t3s, da, db, b3):
    def body(a0_ref, a1_ref, t_ref, da_ref, db_ref, b3_ref, o_ref):
        di = _dinv(da_ref[...], db_ref[...])
        o_ref[...] = (a0_ref[...] + a1_ref[...] + t_ref[...]) * di + b3_ref[...]

    return pl.pallas_call(
        body,
        grid=(_N // _BLK,),
        in_specs=[_row_spec(_C), _row_spec(_C), _row_spec(_C),
                  _row_spec(16), _row_spec(16), _full_spec(1, _C)],
        out_specs=_row_spec(_C),
        out_shape=jax.ShapeDtypeStruct((_N, _C), jnp.float32),
    )(a0, a1, t3s, da, db, b3.reshape(1, _C))


def kernel(x, edge_index, W1, b1, W2, b2, W3, b3):
    edges = edge_index.astype(jnp.int32).reshape(2, _ECH, _CHUNK)
    npad = _PCH * _CHUNK
    pads = jnp.stack([
        jnp.zeros((npad,), jnp.int32),
        _N + (jnp.arange(npad, dtype=jnp.int32) % 8),
    ]).reshape(2, _PCH, _CHUNK)

    ones_blk = jnp.ones((_CHUNK, 16), jnp.float32)
    z16 = jnp.zeros((_N, 16), jnp.float32)
    z32 = jnp.zeros((_N, _H2), jnp.float32)

    degk = _make_deg()
    agg16 = _make_agg(16)
    agg32 = _make_agg(_H2)
    agg64 = _make_agg_cols()

    da, db = degk(ones_blk, edges, pads, z16)
    xs = _k_scale(x, da, db)
    ax = agg64(xs, edges, pads, z32)
    t2s = _k_layer12(ax, xs, da, db, W1, b1, W2)
    c0, c1 = agg32(t2s, edges, pads, z32)
    t3s = _k_layer23(c0, c1, t2s, da, db, b2, W3)
    d0, d1 = agg16(t3s, edges, pads, z16)
    return _k_out(d0, d1, t3s, da, db, b3)


# speedup vs baseline: 35.5197x; 1.1032x over previous
import functools

import jax
import jax.numpy as jnp
from jax import lax
from jax.experimental import pallas as pl
from jax.experimental.pallas import tpu as pltpu
from jax.experimental.pallas import tpu_sc as plsc

_N = 10000
_E = 320000
_D_IN = 128
_DQ = _D_IN // 4
_H1 = 256
_H2 = 32
_C = 16

_NC = 2
_NS = 16
_NW = _NC * _NS
_CHUNK = 128
_CPW = 80
_CPS = 160
_EPAD = _CPW * _CHUNK * _NW
_RPS = 624
_TAIL = _N - _NS * _RPS

_BLK = 1000

_ECH = _E // _CHUNK
_PCH = _EPAD // _CHUNK - _ECH

_SC_PARAMS = pltpu.CompilerParams(use_tc_tiling_on_sc=False)


def _mesh():
    return plsc.VectorSubcoreMesh(core_axis_name="c", subcore_axis_name="s")


def _load_idx(echunks, pchunks, idx_v, base, count):
    nreal = count - _PCH

    @pl.when(base + count <= _ECH)
    def _():
        pltpu.sync_copy(echunks.at[pl.ds(base, count)], idx_v)

    @pl.when(base + count > _ECH)
    def _():
        pltpu.sync_copy(echunks.at[pl.ds(base, nreal)],
                        idx_v.at[pl.ds(0, nreal)])
        pltpu.sync_copy(pchunks, idx_v.at[pl.ds(nreal, _PCH)])


def _init_acc(zeros, acc, s, sl, tl):
    pltpu.sync_copy(zeros.at[sl], acc.at[sl])

    @pl.when(s == _NS - 1)
    def _():
        pltpu.sync_copy(zeros.at[tl], acc.at[tl])


def _pipelined_edge_loop(vals, sidx, didx, rows0, rows1, acc,
                         semg0, semg1, sems0, sems1, cpw):
    plsc.subcore_barrier()
    pltpu.async_copy(vals.at[sidx.at[0]], rows0, semg0)
    pltpu.async_copy(vals.at[sidx.at[1]], rows1, semg1)

    @pl.loop(0, cpw // 2)
    def _(u):
        t0 = 2 * u
        t1 = t0 + 1
        pltpu.make_async_copy(vals.at[sidx.at[t0]], rows0, semg0).wait()
        pltpu.async_copy(rows0, acc.at[didx.at[t0]], sems0, add=True)
        pltpu.make_async_copy(vals.at[sidx.at[t1]], rows1, semg1).wait()
        pltpu.async_copy(rows1, acc.at[didx.at[t1]], sems1, add=True)

        @pl.when(t0 + 2 < cpw)
        def _():
            pltpu.make_async_copy(rows0, acc.at[didx.at[t0]], sems0).wait()
            pltpu.async_copy(vals.at[sidx.at[t0 + 2]], rows0, semg0)
            pltpu.make_async_copy(rows1, acc.at[didx.at[t1]], sems1).wait()
            pltpu.async_copy(vals.at[sidx.at[t1 + 2]], rows1, semg1)

    pltpu.make_async_copy(rows0, acc.at[didx.at[cpw - 2]], sems0).wait()
    pltpu.make_async_copy(rows1, acc.at[didx.at[cpw - 1]], sems1).wait()
    plsc.subcore_barrier()


@functools.cache
def _make_agg(d):
    out = jax.ShapeDtypeStruct((_N, d), jnp.float32)

    @functools.partial(
        pl.kernel,
        out_type=(out, out),
        mesh=_mesh(),
        compiler_params=_SC_PARAMS,
        scratch_types=[
            pltpu.VMEM((_CPW, _CHUNK), jnp.int32),
            pltpu.VMEM((_CPW, _CHUNK), jnp.int32),
            pltpu.VMEM((_CHUNK, d), jnp.float32),
            pltpu.VMEM((_CHUNK, d), jnp.float32),
            pltpu.VMEM_SHARED((_N + 8, d), jnp.float32),
            pltpu.VMEM_SHARED((_N + 8, d), jnp.float32),
            pltpu.SemaphoreType.DMA,
            pltpu.SemaphoreType.DMA,
            pltpu.SemaphoreType.DMA,
            pltpu.SemaphoreType.DMA,
        ],
    )
    def agg(vals, edges, pads, zeros, out0, out1,
            sidx, didx, rows0, rows1, acc, srcbuf, semg0, semg1, sems0, sems1):
        c = lax.axis_index("c")
        s = lax.axis_index("s")
        w = c * _NS + s
        sl = pl.ds(s * _RPS, _RPS)
        tl = pl.ds(_NS * _RPS, _TAIL)
        _init_acc(zeros, acc, s, sl, tl)
        pltpu.sync_copy(vals.at[sl], srcbuf.at[sl])

        @pl.when(s == _NS - 1)
        def _():
            pltpu.sync_copy(vals.at[tl], srcbuf.at[tl])

        _load_idx(edges.at[0], pads.at[0], sidx, w * _CPW, _CPW)
        _load_idx(edges.at[1], pads.at[1], didx, w * _CPW, _CPW)
        _pipelined_edge_loop(srcbuf, sidx, didx, rows0, rows1, acc,
                             semg0, semg1, sems0, sems1, _CPW)

        @pl.when(c == 0)
        def _():
            pltpu.sync_copy(acc.at[sl], out0.at[sl])

            @pl.when(s == _NS - 1)
            def _():
                pltpu.sync_copy(acc.at[tl], out0.at[tl])

        @pl.when(c == 1)
        def _():
            pltpu.sync_copy(acc.at[sl], out1.at[sl])

            @pl.when(s == _NS - 1)
            def _():
                pltpu.sync_copy(acc.at[tl], out1.at[tl])

    return agg


def _make_deg():
    out = jax.ShapeDtypeStruct((_N, 16), jnp.float32)

    @functools.partial(
        pl.kernel,
        out_type=(out, out),
        mesh=_mesh(),
        compiler_params=_SC_PARAMS,
        scratch_types=[
            pltpu.VMEM((_CPW, _CHUNK), jnp.int32),
            pltpu.VMEM((_CHUNK, 16), jnp.float32),
            pltpu.VMEM_SHARED((_N + 8, 16), jnp.float32),
            pltpu.SemaphoreType.DMA,
        ],
    )
    def deg(ones_hbm, edges, pads, zeros, out0, out1, didx, ones, acc, sem):
        c = lax.axis_index("c")
        s = lax.axis_index("s")
        w = c * _NS + s
        sl = pl.ds(s * _RPS, _RPS)
        tl = pl.ds(_NS * _RPS, _TAIL)
        _init_acc(zeros, acc, s, sl, tl)
        pltpu.sync_copy(ones_hbm, ones)
        _load_idx(edges.at[1], pads.at[1], didx, w * _CPW, _CPW)
        plsc.subcore_barrier()

        @pl.loop(0, _CPW // 8)
        def _(g):
            base = g * 8
            for j in range(8):
                pltpu.async_copy(ones, acc.at[didx.at[base + j]], sem, add=True)
            for j in range(8):
                pltpu.make_async_copy(ones, acc.at[didx.at[base + j]], sem).wait()

        plsc.subcore_barrier()

        @pl.when(c == 0)
        def _():
            pltpu.sync_copy(acc.at[sl], out0.at[sl])

            @pl.when(s == _NS - 1)
            def _():
                pltpu.sync_copy(acc.at[tl], out0.at[tl])

        @pl.when(c == 1)
        def _():
            pltpu.sync_copy(acc.at[sl], out1.at[sl])

            @pl.when(s == _NS - 1)
            def _():
                pltpu.sync_copy(acc.at[tl], out1.at[tl])

    return deg


def _make_agg_cols():
    out = jax.ShapeDtypeStruct((_N, _D_IN), jnp.float32)

    @functools.partial(
        pl.kernel,
        out_type=out,
        mesh=_mesh(),
        compiler_params=_SC_PARAMS,
        scratch_types=[
            pltpu.VMEM((_CPS, _CHUNK), jnp.int32),
            pltpu.VMEM((_CPS, _CHUNK), jnp.int32),
            pltpu.VMEM((_CHUNK, _DQ), jnp.float32),
            pltpu.VMEM((_CHUNK, _DQ), jnp.float32),
            pltpu.VMEM_SHARED((_N + 8, _DQ), jnp.float32),
            pltpu.VMEM_SHARED((_N + 8, _DQ), jnp.float32),
            pltpu.SemaphoreType.DMA,
            pltpu.SemaphoreType.DMA,
            pltpu.SemaphoreType.DMA,
            pltpu.SemaphoreType.DMA,
        ],
    )
    def agg(vals, edges, pads, zeros, outx,
            sidx, didx, rows0, rows1, acc, srcbuf, semg0, semg1, sems0, sems1):
        c = lax.axis_index("c")
        s = lax.axis_index("s")
        sl = pl.ds(s * _RPS, _RPS)
        tl = pl.ds(_NS * _RPS, _TAIL)
        _load_idx(edges.at[0], pads.at[0], sidx, s * _CPS, _CPS)
        _load_idx(edges.at[1], pads.at[1], didx, s * _CPS, _CPS)
        for p in range(2):
            q = c * 2 + p
            cq = pl.ds(q * _DQ, _DQ)
            _init_acc(zeros, acc, s, sl, tl)
            pltpu.sync_copy(vals.at[sl, cq], srcbuf.at[sl])

            @pl.when(s == _NS - 1)
            def _():
                pltpu.sync_copy(vals.at[tl, cq], srcbuf.at[tl])

            _pipelined_edge_loop(srcbuf, sidx, didx, rows0, rows1, acc,
                                 semg0, semg1, sems0, sems1, _CPS)
            pltpu.sync_copy(acc.at[sl], outx.at[sl, cq])

            @pl.when(s == _NS - 1)
            def _():
                pltpu.sync_copy(acc.at[tl], outx.at[tl, cq])

    return agg


def _dinv(da, db):
    return lax.rsqrt(da[:, :1] + db[:, :1] + 1.0)


def _row_spec(d):
    return pl.BlockSpec((_BLK, d), lambda i: (i, 0))


def _quarter_spec():
    return pl.BlockSpec((4, _BLK, _DQ), lambda i: (0, i, 0))


def _full_spec(r, c):
    return pl.BlockSpec((r, c), lambda i: (0, 0))


def _k_scale(x, da, db):
    def body(x_ref, da_ref, db_ref, o_ref):
        o_ref[...] = x_ref[...] * _dinv(da_ref[...], db_ref[...])

    return pl.pallas_call(
        body,
        grid=(_N // _BLK,),
        in_specs=[_row_spec(_D_IN), _row_spec(16), _row_spec(16)],
        out_specs=_row_spec(_D_IN),
        out_shape=jax.ShapeDtypeStruct((_N, _D_IN), jnp.float32),
    )(x, da, db)


def _k_layer12(ax, xs, da, db, W1, b1, W2):
    def body(ax_ref, xs_ref, da_ref, db_ref, w1_ref, b1_ref, w2_ref, o_ref):
        di = _dinv(da_ref[...], db_ref[...])
        agg = (ax_ref[...] + xs_ref[...]) * di
        h = jnp.dot(agg, w1_ref[...], preferred_element_type=jnp.float32)
        h = jnp.maximum(h + b1_ref[...], 0.0)
        o_ref[...] = jnp.dot(h, w2_ref[...], preferred_element_type=jnp.float32) * di

    return pl.pallas_call(
        body,
        grid=(_N // _BLK,),
        in_specs=[_row_spec(_D_IN), _row_spec(_D_IN),
                  _row_spec(16), _row_spec(16),
                  _full_spec(_D_IN, _H1), _full_spec(1, _H1),
                  _full_spec(_H1, _H2)],
        out_specs=_row_spec(_H2),
        out_shape=jax.ShapeDtypeStruct((_N, _H2), jnp.float32),
    )(ax, xs, da, db, W1, b1.reshape(1, _H1), W2)


def _k_layer23(a0, a1, t2s, da, db, b2, W3):
    def body(a0_ref, a1_ref, t_ref, da_ref, db_ref, b2_ref, w3_ref, o_ref):
        di = _dinv(da_ref[...], db_ref[...])
        h = (a0_ref[...] + a1_ref[...] + t_ref[...]) * di + b2_ref[...]
        h = jnp.maximum(h, 0.0)
        o_ref[...] = jnp.dot(h, w3_ref[...], preferred_element_type=jnp.float32) * di

    return pl.pallas_call(
        body,
        grid=(_N // _BLK,),
        in_specs=[_row_spec(_H2), _row_spec(_H2), _row_spec(_H2),
                  _row_spec(16), _row_spec(16),
                  _full_spec(1, _H2), _full_spec(_H2, _C)],
        out_specs=_row_spec(_C),
        out_shape=jax.ShapeDtypeStruct((_N, _C), jnp.float32),
    )(a0, a1, t2s, da, db, b2.reshape(1, _H2), W3)


def _k_out(a0, a1, t3s, da, db, b3):
    def body(a0_ref, a1_ref, t_ref, da_ref, db_ref, b3_ref, o_ref):
        di = _dinv(da_ref[...], db_ref[...])
        o_ref[...] = (a0_ref[...] + a1_ref[...] + t_ref[...]) * di + b3_ref[...]

    return pl.pallas_call(
        body,
        grid=(_N // _BLK,),
        in_specs=[_row_spec(_C), _row_spec(_C), _row_spec(_C),
                  _row_spec(16), _row_spec(16), _full_spec(1, _C)],
        out_specs=_row_spec(_C),
        out_shape=jax.ShapeDtypeStruct((_N, _C), jnp.float32),
    )(a0, a1, t3s, da, db, b3.reshape(1, _C))


def kernel(x, edge_index, W1, b1, W2, b2, W3, b3):
    edges = edge_index.astype(jnp.int32).reshape(2, _ECH, _CHUNK)
    npad = _PCH * _CHUNK
    pads = jnp.stack([
        jnp.zeros((npad,), jnp.int32),
        _N + (jnp.arange(npad, dtype=jnp.int32) % 8),
    ]).reshape(2, _PCH, _CHUNK)

    ones_blk = jnp.ones((_CHUNK, 16), jnp.float32)
    z16 = jnp.zeros((_N, 16), jnp.float32)
    z32 = jnp.zeros((_N, _H2), jnp.float32)

    degk = _make_deg()
    agg16 = _make_agg(16)
    agg32 = _make_agg(_H2)
    agg64 = _make_agg_cols()

    da, db = degk(ones_blk, edges, pads, z16)
    xs = _k_scale(x, da, db)
    ax = agg64(xs, edges, pads, z32)
    t2s = _k_layer12(ax, xs, da, db, W1, b1, W2)
    c0, c1 = agg32(t2s, edges, pads, z32)
    t3s = _k_layer23(c0, c1, t2s, da, db, b2, W3)
    d0, d1 = agg16(t3s, edges, pads, z16)
    return _k_out(d0, d1, t3s, da, db, b3)
